# baseline (speedup 1.0000x reference)
import functools

import jax
import jax.numpy as jnp
from jax import lax
from jax.experimental import pallas as pl
from jax.experimental.pallas import tpu as pltpu
from jax.experimental.pallas import tpu_sc as plsc

_NC = 2
_NS = 16
_NW = _NC * _NS
_LANES = 16


def _build_sc_gather(b, n_cond, n_col, d):
    rows = n_cond * n_col
    jrows = rows // 128
    b_per_w = b // _NW
    chunks = rows // _LANES
    cpj = 128 // _LANES

    mesh = plsc.VectorSubcoreMesh(core_axis_name="c", subcore_axis_name="s")

    @functools.partial(
        pl.kernel,
        out_type=jax.ShapeDtypeStruct((b, rows, d), jnp.float32),
        mesh=mesh,
        scratch_types=[
            pltpu.VMEM((jrows, 128), jnp.int32),
            pltpu.VMEM((jrows, 128), jnp.int32),
            pltpu.VMEM((rows, d), jnp.float32),
            pltpu.SemaphoreType.DMA,
        ],
        compiler_params=pltpu.CompilerParams(use_tc_tiling_on_sc=False),
    )
    def sc_gather(m_hbm, idx_hbm, out_hbm, idx_base, idx_b, buf, sem):
        wid = lax.axis_index("s") * _NC + lax.axis_index("c")
        pltpu.sync_copy(idx_hbm, idx_base)

        def per_batch(bi, carry):
            bb = wid * b_per_w + bi
            off = bb * rows

            def add_chunk(t, c2):
                j = t // cpj
                sl = pl.ds((t % cpj) * _LANES, _LANES)
                idx_b[j, sl] = idx_base[j, sl] + off
                return c2

            lax.fori_loop(0, chunks, add_chunk, 0)

            def fire(j, c2):
                pltpu.async_copy(
                    m_hbm.at[idx_b.at[j]], buf.at[pl.ds(j * 128, 128)], sem
                )
                return c2

            lax.fori_loop(0, jrows, fire, 0)
            pltpu.make_async_copy(out_hbm.at[bb], buf, sem).wait()
            pltpu.sync_copy(buf, out_hbm.at[bb])
            return carry

        lax.fori_loop(0, b_per_w, per_batch, 0)

    return sc_gather


def kernel(M, permutator):
    b, n_cond, n_col, d = M.shape
    rows = n_cond * n_col
    idx_local = (
        permutator.T.astype(jnp.int32) * n_col
        + jnp.arange(n_col, dtype=jnp.int32)[None, :]
    ).reshape(rows // 128, 128)
    out = _build_sc_gather(b, n_cond, n_col, d)(M.reshape(b * rows, d), idx_local)
    return out.reshape(b, n_cond, n_col, d)


# speedup vs baseline: 3.8502x; 3.8502x over previous
import functools

import jax
import jax.numpy as jnp
from jax import lax
from jax.experimental import pallas as pl
from jax.experimental.pallas import tpu as pltpu
from jax.experimental.pallas import tpu_sc as plsc

_NC = 2
_NS = 16
_NW = _NC * _NS
_LANES = 16


def _build_sc_permute(b, n_cond, n_col, d):
    cd = n_col * d
    ncg = cd // 128
    nbgrp = _NW // ncg
    bper = b // nbgrp
    blk = n_cond * 128
    chunks = blk // _LANES
    cpr = 128 // _LANES

    mesh = plsc.VectorSubcoreMesh(core_axis_name="c", subcore_axis_name="s")

    @functools.partial(
        pl.kernel,
        out_type=jax.ShapeDtypeStruct((b, n_cond, cd), jnp.float32),
        mesh=mesh,
        scratch_types=[
            pltpu.VMEM((blk,), jnp.int32),
            pltpu.VMEM((2, n_cond, 128), jnp.float32),
            pltpu.VMEM((2, n_cond, 128), jnp.float32),
            pltpu.SemaphoreType.DMA,
            pltpu.SemaphoreType.DMA,
        ],
        compiler_params=pltpu.CompilerParams(
            use_tc_tiling_on_sc=False, needs_layout_passes=False
        ),
    )
    def sc_permute(m_hbm, idx_hbm, out_hbm, idx_v, in_v, out_v, s_in, s_out):
        wid = lax.axis_index("s") * _NC + lax.axis_index("c")
        cg = lax.rem(wid, ncg)
        bbase = (wid // ncg) * bper
        cgs = pl.ds(cg * 128, 128)
        pltpu.sync_copy(idx_hbm.at[cg], idx_v)

        pltpu.async_copy(m_hbm.at[bbase, :, cgs], in_v.at[0], s_in)

        def task(t, carry):
            s = lax.rem(t, 2)
            bb = bbase + t
            pltpu.make_async_copy(m_hbm.at[bb, :, cgs], in_v.at[s], s_in).wait()

            @pl.when(t + 1 < bper)
            def _prefetch():
                pltpu.async_copy(
                    m_hbm.at[bb + 1, :, cgs], in_v.at[1 - s], s_in
                )

            @pl.when(t >= 2)
            def _drain_one():
                pltpu.make_async_copy(
                    m_hbm.at[bb, :, cgs], out_v.at[s], s_out
                ).wait()

            def chunk(q, c2):
                iv = idx_v[pl.ds(q * _LANES, _LANES)]
                r = iv >> 7
                cc = iv & 127
                out_v[s, q // cpr, pl.ds((q % cpr) * _LANES, _LANES)] = (
                    plsc.load_gather(in_v.at[s], [r, cc])
                )
                return c2

            lax.fori_loop(0, chunks, chunk, 0)
            pltpu.async_copy(out_v.at[s], out_hbm.at[bb, :, cgs], s_out)
            return carry

        lax.fori_loop(0, bper, task, 0)
        pltpu.make_async_copy(m_hbm.at[bbase, :, cgs], out_v.at[0], s_out).wait()
        pltpu.make_async_copy(m_hbm.at[bbase, :, cgs], out_v.at[1], s_out).wait()

    return sc_permute


def kernel(M, permutator):
    b, n_cond, n_col, d = M.shape
    cd = n_col * d
    ncg = cd // 128
    cpg = 128 // d
    permT = permutator.T.astype(jnp.int32)
    idx = (
        permT.reshape(n_cond, ncg, cpg).transpose(1, 0, 2)[..., None] * 128
        + (jnp.arange(cpg, dtype=jnp.int32) * d)[None, None, :, None]
        + jnp.arange(d, dtype=jnp.int32)[None, None, None, :]
    ).reshape(ncg, n_cond * 128)
    out = _build_sc_permute(b, n_cond, n_col, d)(
        M.reshape(b, n_cond, cd), idx
    )
    return out.reshape(b, n_cond, n_col, d)


# speedup vs baseline: 7.1149x; 1.8479x over previous
import functools

import jax
import jax.numpy as jnp
from jax import lax
from jax.experimental import pallas as pl
from jax.experimental.pallas import tpu as pltpu
from jax.experimental.pallas import tpu_sc as plsc

_NC = 2
_NS = 16
_NW = _NC * _NS
_LANES = 16


def _build_sc_permute(b, n_cond, n_col, d):
    cd = n_col * d
    ncg = cd // 128
    nbgrp = _NW // ncg
    bper = b // nbgrp
    blk = n_cond * 128
    chunks = blk // _LANES
    cpr = 128 // _LANES

    mesh = plsc.VectorSubcoreMesh(core_axis_name="c", subcore_axis_name="s")

    @functools.partial(
        pl.kernel,
        out_type=jax.ShapeDtypeStruct((b, n_cond, cd), jnp.float32),
        mesh=mesh,
        scratch_types=[
            pltpu.VMEM((blk,), jnp.int32),
            pltpu.VMEM((2, n_cond, 128), jnp.float32),
            pltpu.VMEM((2, n_cond, 128), jnp.float32),
            pltpu.SemaphoreType.DMA,
            pltpu.SemaphoreType.DMA,
        ],
        compiler_params=pltpu.CompilerParams(
            use_tc_tiling_on_sc=False, needs_layout_passes=False
        ),
    )
    def sc_permute(m_hbm, idx_hbm, out_hbm, idx_v, in_v, out_v, s_in, s_out):
        wid = lax.axis_index("s") * _NC + lax.axis_index("c")
        cg = lax.rem(wid, ncg)
        bbase = (wid // ncg) * bper
        cgs = pl.ds(cg * 128, 128)
        pltpu.sync_copy(idx_hbm.at[cg], idx_v)

        pltpu.async_copy(m_hbm.at[bbase, :, cgs], in_v.at[0], s_in)

        def task(t, carry):
            s = lax.rem(t, 2)
            bb = bbase + t
            pltpu.make_async_copy(m_hbm.at[bb, :, cgs], in_v.at[s], s_in).wait()

            @pl.when(t + 1 < bper)
            def _prefetch():
                pltpu.async_copy(
                    m_hbm.at[bb + 1, :, cgs], in_v.at[1 - s], s_in
                )

            @pl.when(t >= 2)
            def _drain_one():
                pltpu.make_async_copy(
                    m_hbm.at[bb, :, cgs], out_v.at[s], s_out
                ).wait()

            @plsc.parallel_loop(0, chunks, unroll=8)
            def _chunk(q):
                iv = idx_v[pl.ds(q * _LANES, _LANES)]
                r = iv >> 7
                cc = iv & 127
                out_v[s, q // cpr, pl.ds((q % cpr) * _LANES, _LANES)] = (
                    plsc.load_gather(in_v.at[s], [r, cc])
                )
            pltpu.async_copy(out_v.at[s], out_hbm.at[bb, :, cgs], s_out)
            return carry

        lax.fori_loop(0, bper, task, 0)
        pltpu.make_async_copy(m_hbm.at[bbase, :, cgs], out_v.at[0], s_out).wait()
        pltpu.make_async_copy(m_hbm.at[bbase, :, cgs], out_v.at[1], s_out).wait()

    return sc_permute


def kernel(M, permutator):
    b, n_cond, n_col, d = M.shape
    cd = n_col * d
    ncg = cd // 128
    cpg = 128 // d
    permT = permutator.T.astype(jnp.int32)
    idx = (
        permT.reshape(n_cond, ncg, cpg).transpose(1, 0, 2)[..., None] * 128
        + (jnp.arange(cpg, dtype=jnp.int32) * d)[None, None, :, None]
        + jnp.arange(d, dtype=jnp.int32)[None, None, None, :]
    ).reshape(ncg, n_cond * 128)
    out = _build_sc_permute(b, n_cond, n_col, d)(
        M.reshape(b, n_cond, cd), idx
    )
    return out.reshape(b, n_cond, n_col, d)


# speedup vs baseline: 31.1056x; 4.3719x over previous
import functools

import jax
import jax.numpy as jnp
from jax import lax
from jax.experimental import pallas as pl
from jax.experimental.pallas import tpu as pltpu
from jax.experimental.pallas import tpu_sc as plsc

_NC = 2
_NS = 16
_NW = _NC * _NS
_LANES = 16


def _build_sc_permute(b, n_cond, n_col, d):
    rows_total = b * n_col * d
    rpb = 128
    rpw = rows_total // _NW
    ntasks = rpw // rpb
    chunks = rpb * n_cond // _LANES
    cpr = n_cond // _LANES
    ncg = (n_col * d) // rpb

    mesh = plsc.VectorSubcoreMesh(core_axis_name="c", subcore_axis_name="s")

    @functools.partial(
        pl.kernel,
        out_type=jax.ShapeDtypeStruct((rows_total, n_cond), jnp.float32),
        mesh=mesh,
        scratch_types=[
            pltpu.VMEM((n_col, n_cond), jnp.int32),
            pltpu.VMEM((2, rpb, n_cond), jnp.float32),
            pltpu.VMEM((2, rpb, n_cond), jnp.float32),
            pltpu.SemaphoreType.DMA,
            pltpu.SemaphoreType.DMA,
        ],
        compiler_params=pltpu.CompilerParams(needs_layout_passes=False),
    )
    def sc_permute(m_hbm, idx_hbm, out_hbm, idx_v, in_v, out_v, s_in, s_out):
        wid = lax.axis_index("s") * _NC + lax.axis_index("c")
        base = wid * rpw
        pltpu.sync_copy(idx_hbm, idx_v)

        pltpu.async_copy(m_hbm.at[pl.ds(base, rpb), :], in_v.at[0], s_in)

        def task(t, carry):
            s = lax.rem(t, 2)
            r0 = base + t * rpb
            rsl = pl.ds(r0, rpb)
            pltpu.make_async_copy(m_hbm.at[rsl, :], in_v.at[s], s_in).wait()

            @pl.when(t + 1 < ntasks)
            def _prefetch():
                pltpu.async_copy(
                    m_hbm.at[pl.ds(r0 + rpb, rpb), :], in_v.at[1 - s], s_in
                )

            @pl.when(t >= 2)
            def _drain_one():
                pltpu.make_async_copy(
                    m_hbm.at[rsl, :], out_v.at[s], s_out
                ).wait()

            cbase = lax.rem(t, ncg) * (n_col // ncg)

            @plsc.parallel_loop(0, chunks, unroll=8)
            def _chunk(q):
                row = q // cpr
                lsl = pl.ds(lax.rem(q, cpr) * _LANES, _LANES)
                iv = idx_v[cbase + q // (cpr * d), lsl]
                r = lax.broadcast(row, (_LANES,))
                out_v[s, row, lsl] = plsc.load_gather(in_v.at[s], [r, iv])

            pltpu.async_copy(out_v.at[s], out_hbm.at[rsl, :], s_out)
            return carry

        lax.fori_loop(0, ntasks, task, 0)
        bsl = pl.ds(base, rpb)
        pltpu.make_async_copy(m_hbm.at[bsl, :], out_v.at[0], s_out).wait()
        pltpu.make_async_copy(m_hbm.at[bsl, :], out_v.at[1], s_out).wait()

    return sc_permute


def kernel(M, permutator):
    b, n_cond, n_col, d = M.shape
    mp = jnp.transpose(M, (0, 2, 3, 1)).reshape(b * n_col * d, n_cond)
    out = _build_sc_permute(b, n_cond, n_col, d)(
        mp, permutator.astype(jnp.int32)
    )
    return out.reshape(b, n_col, d, n_cond).transpose(0, 3, 1, 2)


# speedup vs baseline: 31.9774x; 1.0280x over previous
import functools

import jax
import jax.numpy as jnp
from jax import lax
from jax.experimental import pallas as pl
from jax.experimental.pallas import tpu as pltpu
from jax.experimental.pallas import tpu_sc as plsc

_NC = 2
_NS = 16
_NW = _NC * _NS
_LANES = 16


def _build_sc_permute(b, n_cond, n_col, d):
    rows_total = b * n_col * d
    rpb = 128
    rpw = rows_total // _NW
    ntasks = rpw // rpb
    cpr = n_cond // _LANES
    ncg = (n_col * d) // rpb
    cpg = rpb // d

    mesh = plsc.VectorSubcoreMesh(core_axis_name="c", subcore_axis_name="s")

    @functools.partial(
        pl.kernel,
        out_type=jax.ShapeDtypeStruct((rows_total, n_cond), jnp.float32),
        mesh=mesh,
        scratch_types=[
            pltpu.VMEM((n_col, n_cond), jnp.int32),
            pltpu.VMEM((2, rpb, n_cond), jnp.float32),
            pltpu.VMEM((2, rpb, n_cond), jnp.float32),
            pltpu.SemaphoreType.DMA,
            pltpu.SemaphoreType.DMA,
        ],
        compiler_params=pltpu.CompilerParams(needs_layout_passes=False),
    )
    def sc_permute(m_hbm, idx_hbm, out_hbm, idx_v, in_v, out_v, s_in, s_out):
        wid = lax.axis_index("s") * _NC + lax.axis_index("c")
        base = wid * rpw
        pltpu.sync_copy(idx_hbm, idx_v)

        pltpu.async_copy(m_hbm.at[pl.ds(base, rpb), :], in_v.at[0], s_in)

        def task(t, carry):
            s = lax.rem(t, 2)
            r0 = base + t * rpb
            rsl = pl.ds(r0, rpb)
            pltpu.make_async_copy(m_hbm.at[rsl, :], in_v.at[s], s_in).wait()

            @pl.when(t + 1 < ntasks)
            def _prefetch():
                pltpu.async_copy(
                    m_hbm.at[pl.ds(r0 + rpb, rpb), :], in_v.at[1 - s], s_in
                )

            @pl.when(t >= 2)
            def _drain_one():
                pltpu.make_async_copy(
                    m_hbm.at[rsl, :], out_v.at[s], s_out
                ).wait()

            cbase = lax.rem(t, ncg) * cpg

            def col_body(cl, c2):
                ivs = [
                    idx_v[cbase + cl, pl.ds(g * _LANES, _LANES)]
                    for g in range(cpr)
                ]

                @plsc.parallel_loop(0, d, unroll=d)
                def _row(rr):
                    row = cl * d + rr
                    rspl = lax.broadcast(row, (_LANES,))
                    for g in range(cpr):
                        out_v[s, row, pl.ds(g * _LANES, _LANES)] = (
                            plsc.load_gather(in_v.at[s], [rspl, ivs[g]])
                        )

                return c2

            lax.fori_loop(0, cpg, col_body, 0)

            pltpu.async_copy(out_v.at[s], out_hbm.at[rsl, :], s_out)
            return carry

        lax.fori_loop(0, ntasks, task, 0)
        bsl = pl.ds(base, rpb)
        pltpu.make_async_copy(m_hbm.at[bsl, :], out_v.at[0], s_out).wait()
        pltpu.make_async_copy(m_hbm.at[bsl, :], out_v.at[1], s_out).wait()

    return sc_permute


def kernel(M, permutator):
    b, n_cond, n_col, d = M.shape
    mp = jnp.transpose(M, (0, 2, 3, 1)).reshape(b * n_col * d, n_cond)
    out = _build_sc_permute(b, n_cond, n_col, d)(
        mp, permutator.astype(jnp.int32)
    )
    return out.reshape(b, n_col, d, n_cond).transpose(0, 3, 1, 2)
